# Initial kernel scaffold; baseline (speedup 1.0000x reference)
import jax
import jax.numpy as jnp
from jax.experimental import pallas as pl


def kernel(indices, table):
    raise NotImplementedError("write your pallas kernel here")


# speedup vs baseline: 4.7591x; 4.7591x over previous
import functools

import jax
import jax.numpy as jnp
from jax import lax
from jax.experimental import pallas as pl
from jax.experimental.pallas import tpu as pltpu
from jax.experimental.pallas import tpu_sc as plsc

B = 4096
L = 200
S = 3
D = 64
N = B * L
NC = 2
NS = 16
NW = NC * NS
TOK_PER_W = N // NW
CHUNK = 128
NCHUNK = TOK_PER_W // CHUNK
NHALF = NCHUNK // 2


def _embed_sum(table_hbm, idx_hbm, out_hbm, idx_v, rows_v, out_v, gsem0, gsem1):
    wid = lax.axis_index("s") * NC + lax.axis_index("c")
    base0 = wid * TOK_PER_W
    gsems = (gsem0, gsem1)

    def start(g, p):
        tok = base0 + g * CHUNK
        pltpu.sync_copy(idx_hbm.at[:, pl.ds(tok, CHUNK)], idx_v.at[p])
        pltpu.async_copy(table_hbm.at[idx_v.at[p, 0]], out_v.at[p], gsems[p])
        pltpu.async_copy(table_hbm.at[idx_v.at[p, 1]], rows_v.at[p, 0], gsems[p])
        pltpu.async_copy(table_hbm.at[idx_v.at[p, 2]], rows_v.at[p, 1], gsems[p])

    def finish(g, p):
        pltpu.make_async_copy(table_hbm.at[idx_v.at[p, 0]], out_v.at[p], gsems[p]).wait()
        pltpu.make_async_copy(table_hbm.at[idx_v.at[p, 1]], rows_v.at[p, 0], gsems[p]).wait()
        pltpu.make_async_copy(table_hbm.at[idx_v.at[p, 2]], rows_v.at[p, 1], gsems[p]).wait()

        def cbody(t, carry):
            for j in range(D // 16):
                sl = pl.ds(j * 16, 16)
                a = rows_v[p, 0, t, sl] + rows_v[p, 1, t, sl]
                plsc.addupdate(out_v.at[p, t, sl], a)
            return carry

        lax.fori_loop(0, CHUNK, cbody, 0, unroll=2)
        tok = base0 + g * CHUNK
        pltpu.sync_copy(out_v.at[p], out_hbm.at[pl.ds(tok, CHUNK)])

    start(0, 0)

    def body(i, carry):
        g0 = 2 * i
        start(g0 + 1, 1)
        finish(g0, 0)

        @pl.when(i < NHALF - 1)
        def _():
            start(g0 + 2, 0)

        finish(g0 + 1, 1)
        return carry

    lax.fori_loop(0, NHALF, body, 0)


@jax.jit
def _call(table, idx_t):
    mesh = plsc.VectorSubcoreMesh(core_axis_name="c", subcore_axis_name="s")
    run = functools.partial(
        pl.kernel,
        out_type=jax.ShapeDtypeStruct((N, D), jnp.float32),
        mesh=mesh,
        compiler_params=pltpu.CompilerParams(use_tc_tiling_on_sc=False),
        scratch_types=[
            pltpu.VMEM((2, S, CHUNK), jnp.int32),
            pltpu.VMEM((2, 2, CHUNK, D), jnp.float32),
            pltpu.VMEM((2, CHUNK, D), jnp.float32),
            pltpu.SemaphoreType.DMA,
            pltpu.SemaphoreType.DMA,
        ],
    )(_embed_sum)
    return run(table, idx_t)


def kernel(indices, table):
    idx_t = indices.astype(jnp.int32).reshape(N, S).T
    out = _call(table, idx_t)
    return out.reshape(B, L, D)


# speedup vs baseline: 4.8480x; 1.0187x over previous
import functools

import jax
import jax.numpy as jnp
from jax import lax
from jax.experimental import pallas as pl
from jax.experimental.pallas import tpu as pltpu
from jax.experimental.pallas import tpu_sc as plsc

B = 4096
L = 200
S = 3
D = 64
N = B * L
NC = 2
NS = 16
NW = NC * NS
IB = 128
K = 2
CHUNK = K * IB
TOK_PER_W = N // NW
NCHUNK = TOK_PER_W // CHUNK
NBLK = N // IB
UNROLL = 4


def _embed_sum(table_hbm, idx_hbm, out_hbm, idx_v, out_v,
               isem0, isem1, isem2, isem3, gsem0, gsem1, asem0, asem1,
               osem0, osem1):
    wid = lax.axis_index("s") * NC + lax.axis_index("c")
    blk0 = wid * (TOK_PER_W // IB)
    tok0 = wid * TOK_PER_W
    isems = (isem0, isem1, isem2, isem3)
    gsems = (gsem0, gsem1)
    asems = (asem0, asem1)
    osems = (osem0, osem1)

    def idx_copy(c, ph):
        return pltpu.make_async_copy(
            idx_hbm.at[:, pl.ds(blk0 + c * K, K), :], idx_v.at[ph], isems[ph])

    def gath0(c, ph, p):
        return [pltpu.make_async_copy(
                    table_hbm.at[idx_v.at[ph, 0, k]],
                    out_v.at[p, pl.ds(k * IB, IB)], gsems[p])
                for k in range(K)]

    def gath_add_start(c, ph, p):
        for s in (1, 2):
            for k in range(K):
                pltpu.async_copy(
                    table_hbm.at[idx_v.at[ph, s, k]],
                    out_v.at[p, pl.ds(k * IB, IB)], asems[p], add=True)

    def gath_add_wait(c, ph, p):
        for s in (1, 2):
            for k in range(K):
                pltpu.make_async_copy(
                    table_hbm.at[idx_v.at[ph, s, k]],
                    out_v.at[p, pl.ds(k * IB, IB)], asems[p]).wait()

    def out_copy(c, p):
        return pltpu.make_async_copy(
            out_v.at[p], out_hbm.at[pl.ds(tok0 + c * CHUNK, CHUNK)], osems[p])

    idx_copy(0, 0).start()
    idx_copy(1, 1).start()
    idx_copy(0, 0).wait()
    for d in gath0(0, 0, 0):
        d.start()

    def step(c, p, ph, first, last, pf_idx=True):
        q = 1 - p
        phn = (ph + 1) % UNROLL
        if not last:
            if not first:
                out_copy(c - 1, q).wait()
            idx_copy(c + 1, phn).wait()
            for d in gath0(c + 1, phn, q):
                d.start()
        for d in gath0(c, ph, p):
            d.wait()
        gath_add_start(c, ph, p)
        if pf_idx:
            idx_copy(c + 2, (ph + 2) % UNROLL).start()
        gath_add_wait(c, ph, p)
        out_copy(c, p).start()

    def body(i, carry):
        for u in range(UNROLL):
            step(UNROLL * i + u, u % 2, u, False, False)
        return carry

    step(0, 0, 0, True, False)
    step(1, 1, 1, False, False)
    step(2, 0, 2, False, False)
    step(3, 1, 3, False, False)
    lax.fori_loop(1, NCHUNK // UNROLL - 1, body, 0)
    step(NCHUNK - 4, 0, 0, False, False)
    step(NCHUNK - 3, 1, 1, False, False)
    step(NCHUNK - 2, 0, 2, False, False, pf_idx=False)
    step(NCHUNK - 1, 1, 3, False, True, pf_idx=False)
    out_copy(NCHUNK - 2, 0).wait()
    out_copy(NCHUNK - 1, 1).wait()


@jax.jit
def _call(table, idx_r):
    mesh = plsc.VectorSubcoreMesh(core_axis_name="c", subcore_axis_name="s")
    run = functools.partial(
        pl.kernel,
        out_type=jax.ShapeDtypeStruct((N, D), jnp.float32),
        mesh=mesh,
        compiler_params=pltpu.CompilerParams(use_tc_tiling_on_sc=False),
        scratch_types=[
            pltpu.VMEM((UNROLL, S, K, IB), jnp.int32),
            pltpu.VMEM((2, CHUNK, D), jnp.float32),
        ] + [pltpu.SemaphoreType.DMA] * 10,
    )(_embed_sum)
    return run(table, idx_r)


def kernel(indices, table):
    idx_r = indices.astype(jnp.int32).reshape(N, S).T.reshape(S, NBLK, IB)
    out = _call(table, idx_r)
    return out.reshape(B, L, D)
